# Initial kernel scaffold; baseline (speedup 1.0000x reference)
import jax
import jax.numpy as jnp
from jax.experimental import pallas as pl


def kernel(x, codebook):
    raise NotImplementedError("write your pallas kernel here")


# speedup vs baseline: 1.2533x; 1.2533x over previous
import functools

import jax
import jax.numpy as jnp
from jax import lax
from jax.experimental import pallas as pl
from jax.experimental.pallas import tpu as pltpu
from jax.experimental.pallas import tpu_sc as plsc

N = 16384
K = 8192
C = 64
BN = 256


def _bf16_round(v):
    u = lax.bitcast_convert_type(v, jnp.uint32)
    r = (u + jnp.uint32(0x7FFF) + ((u >> 16) & jnp.uint32(1))) & jnp.uint32(0xFFFF0000)
    return lax.bitcast_convert_type(r, jnp.float32)


def _half_argmin(d, base):
    mv = jnp.min(d, axis=1, keepdims=True)
    ids = lax.broadcasted_iota(jnp.int32, d.shape, 1)
    mi = jnp.min(jnp.where(d == mv, ids, K), axis=1, keepdims=True) + base
    return mv, mi


def _argmin_body(x_ref, cb_ref, idx_ref, c2_ref):
    @pl.when(pl.program_id(0) == 0)
    def _():
        cb = cb_ref[...]
        c2_ref[...] = jnp.sum(cb * cb, axis=1)

    x = x_ref[...]
    x2 = jnp.sum(x * x, axis=1, keepdims=True)
    mm = lax.dot_general(x.astype(jnp.bfloat16), cb_ref[...].astype(jnp.bfloat16),
                         (((1,), (1,)), ((), ())),
                         preferred_element_type=jnp.float32)
    dists = x2 - 2.0 * mm + c2_ref[...][None, :]
    mv1, mi1 = _half_argmin(dists[:, : K // 2], 0)
    mv2, mi2 = _half_argmin(dists[:, K // 2 :], K // 2)
    keep = _bf16_round(mv1) <= mv2
    idx_ref[...] = jnp.where(keep, mi1, mi2)[:, 0]


def _argmin_call(x, codebook):
    return pl.pallas_call(
        _argmin_body,
        grid=(N // BN,),
        in_specs=[
            pl.BlockSpec((BN, C), lambda i: (i, 0)),
            pl.BlockSpec((K, C), lambda i: (0, 0)),
        ],
        out_specs=pl.BlockSpec((BN,), lambda i: (i,)),
        out_shape=jax.ShapeDtypeStruct((N,), jnp.int32),
        scratch_shapes=[pltpu.VMEM((K,), jnp.float32)],
    )(x, codebook)


def _make_gather():
    info = plsc.get_sparse_core_info()
    nw = info.num_cores * info.num_subcores
    b_per_w = N // nw
    ch = 128
    nch = b_per_w // ch
    mesh = plsc.VectorSubcoreMesh(core_axis_name="c", subcore_axis_name="s")

    @functools.partial(
        pl.kernel,
        mesh=mesh,
        compiler_params=pltpu.CompilerParams(use_tc_tiling_on_sc=False),
        out_type=jax.ShapeDtypeStruct((nw, nch, ch, C), jnp.float32),
        scratch_types=[
            pltpu.VMEM((nch, ch), jnp.int32),
            pltpu.VMEM((nch, ch, C), jnp.float32),
            pltpu.SemaphoreType.DMA,
        ],
    )
    def gather(cb_hbm, idx_hbm, out_hbm, idx_v, rows_v, sem):
        wid = lax.axis_index("s") * info.num_cores + lax.axis_index("c")
        pltpu.sync_copy(idx_hbm.at[wid], idx_v)
        copies = [
            pltpu.async_copy(cb_hbm.at[idx_v.at[j]], rows_v.at[j], sem)
            for j in range(nch)
        ]
        for cp in copies:
            cp.wait()
        pltpu.sync_copy(rows_v, out_hbm.at[wid])

    return gather, nw, nch, ch


def kernel(x, codebook):
    idx = _argmin_call(x, codebook)
    gather, nw, nch, ch = _make_gather()
    quantized = gather(codebook, idx.reshape(nw, nch, ch))
    return quantized.reshape(N, C), idx


# speedup vs baseline: 1.3131x; 1.0477x over previous
import functools

import jax
import jax.numpy as jnp
from jax import lax
from jax.experimental import pallas as pl
from jax.experimental.pallas import tpu as pltpu
from jax.experimental.pallas import tpu_sc as plsc

N = 16384
K = 8192
C = 64
BN = 256


def _bf16_round(v):
    u = lax.bitcast_convert_type(v, jnp.uint32)
    r = (u + jnp.uint32(0x7FFF) + ((u >> 16) & jnp.uint32(1))) & jnp.uint32(0xFFFF0000)
    return lax.bitcast_convert_type(r, jnp.float32)


def _half_argmin(d, base):
    mv = jnp.min(d, axis=1, keepdims=True)
    ids = lax.broadcasted_iota(jnp.int32, d.shape, 1)
    mi = jnp.min(jnp.where(d == mv, ids, K), axis=1, keepdims=True) + base
    return mv, mi


def _argmin_body(x_ref, cb_ref, idx_ref, c2_ref, cbh_ref):
    @pl.when(pl.program_id(0) == 0)
    def _():
        cb = cb_ref[...]
        c2_ref[...] = jnp.sum(cb * cb, axis=1)
        cbh_ref[...] = cb.astype(jnp.bfloat16).T

    x = x_ref[...]
    x2 = jnp.sum(x * x, axis=1, keepdims=True)
    mm = lax.dot_general(x.astype(jnp.bfloat16), cbh_ref[...],
                         (((1,), (0,)), ((), ())),
                         preferred_element_type=jnp.float32)
    dists = x2 - 2.0 * mm + c2_ref[...][None, :]
    mv1, mi1 = _half_argmin(dists[:, : K // 2], 0)
    mv2, mi2 = _half_argmin(dists[:, K // 2 :], K // 2)
    keep = _bf16_round(mv1) <= mv2
    idx_ref[...] = jnp.where(keep, mi1, mi2)[:, 0]


def _argmin_call(x, codebook):
    return pl.pallas_call(
        _argmin_body,
        grid=(N // BN,),
        in_specs=[
            pl.BlockSpec((BN, C), lambda i: (i, 0)),
            pl.BlockSpec((K, C), lambda i: (0, 0)),
        ],
        out_specs=pl.BlockSpec((BN,), lambda i: (i,)),
        out_shape=jax.ShapeDtypeStruct((N,), jnp.int32),
        scratch_shapes=[pltpu.VMEM((K,), jnp.float32),
                        pltpu.VMEM((C, K), jnp.bfloat16)],
    )(x, codebook)


def _make_gather():
    info = plsc.get_sparse_core_info()
    nw = info.num_cores * info.num_subcores
    b_per_w = N // nw
    ch = 128
    nch = b_per_w // ch
    mesh = plsc.VectorSubcoreMesh(core_axis_name="c", subcore_axis_name="s")

    @functools.partial(
        pl.kernel,
        mesh=mesh,
        compiler_params=pltpu.CompilerParams(use_tc_tiling_on_sc=False),
        out_type=jax.ShapeDtypeStruct((nw, nch, ch, C), jnp.float32),
        scratch_types=[
            pltpu.VMEM((nch, ch), jnp.int32),
            pltpu.VMEM((nch, ch, C), jnp.float32),
            pltpu.SemaphoreType.DMA,
        ],
    )
    def gather(cb_hbm, idx_hbm, out_hbm, idx_v, rows_v, sem):
        wid = lax.axis_index("s") * info.num_cores + lax.axis_index("c")
        pltpu.sync_copy(idx_hbm.at[wid], idx_v)
        copies = [
            pltpu.async_copy(cb_hbm.at[idx_v.at[j]], rows_v.at[j], sem)
            for j in range(nch)
        ]
        for cp in copies:
            cp.wait()
        pltpu.sync_copy(rows_v, out_hbm.at[wid])

    return gather, nw, nch, ch


def kernel(x, codebook):
    idx = _argmin_call(x, codebook)
    gather, nw, nch, ch = _make_gather()
    quantized = gather(codebook, idx.reshape(nw, nch, ch))
    return quantized.reshape(N, C), idx


# speedup vs baseline: 1.3667x; 1.0408x over previous
import functools

import jax
import jax.numpy as jnp
from jax import lax
from jax.experimental import pallas as pl
from jax.experimental.pallas import tpu as pltpu
from jax.experimental.pallas import tpu_sc as plsc

N = 16384
K = 8192
C = 64
BN = 1024


def _bf16_round(v):
    u = lax.bitcast_convert_type(v, jnp.uint32)
    r = (u + jnp.uint32(0x7FFF) + ((u >> 16) & jnp.uint32(1))) & jnp.uint32(0xFFFF0000)
    return lax.bitcast_convert_type(r, jnp.float32)


def _half_argmin(d, base):
    mv = jnp.min(d, axis=1, keepdims=True)
    mi = jnp.argmin(d, axis=1)[:, None] + base
    return mv, mi


def _argmin_body(x_ref, cb_ref, idx_ref, c2_ref, cbh_ref):
    @pl.when(pl.program_id(0) == 0)
    def _():
        cb = cb_ref[...]
        c2_ref[...] = jnp.sum(cb * cb, axis=1)
        cbh_ref[...] = cb.astype(jnp.bfloat16).T

    x = x_ref[...]
    x2 = jnp.sum(x * x, axis=1, keepdims=True)
    mm = lax.dot_general(x.astype(jnp.bfloat16), cbh_ref[...],
                         (((1,), (0,)), ((), ())),
                         preferred_element_type=jnp.float32)
    dists = x2 - 2.0 * mm + c2_ref[...][None, :]
    mv1, mi1 = _half_argmin(dists[:, : K // 2], 0)
    mv2, mi2 = _half_argmin(dists[:, K // 2 :], K // 2)
    keep = _bf16_round(mv1) <= mv2
    idx_ref[...] = jnp.where(keep, mi1, mi2)[:, 0]


def _argmin_call(x, codebook):
    return pl.pallas_call(
        _argmin_body,
        grid=(N // BN,),
        in_specs=[
            pl.BlockSpec((BN, C), lambda i: (i, 0)),
            pl.BlockSpec((K, C), lambda i: (0, 0)),
        ],
        out_specs=pl.BlockSpec((BN,), lambda i: (i,)),
        out_shape=jax.ShapeDtypeStruct((N,), jnp.int32),
        scratch_shapes=[pltpu.VMEM((K,), jnp.float32),
                        pltpu.VMEM((C, K), jnp.bfloat16)],
    )(x, codebook)


def _make_gather():
    info = plsc.get_sparse_core_info()
    nw = info.num_cores * info.num_subcores
    b_per_w = N // nw
    ch = 128
    nch = b_per_w // ch
    mesh = plsc.VectorSubcoreMesh(core_axis_name="c", subcore_axis_name="s")

    @functools.partial(
        pl.kernel,
        mesh=mesh,
        compiler_params=pltpu.CompilerParams(use_tc_tiling_on_sc=False),
        out_type=jax.ShapeDtypeStruct((nw, nch, ch, C), jnp.float32),
        scratch_types=[
            pltpu.VMEM((nch, ch), jnp.int32),
            pltpu.VMEM((nch, ch, C), jnp.float32),
            pltpu.SemaphoreType.DMA,
        ],
    )
    def gather(cb_hbm, idx_hbm, out_hbm, idx_v, rows_v, sem):
        wid = lax.axis_index("s") * info.num_cores + lax.axis_index("c")
        pltpu.sync_copy(idx_hbm.at[wid], idx_v)
        copies = [
            pltpu.async_copy(cb_hbm.at[idx_v.at[j]], rows_v.at[j], sem)
            for j in range(nch)
        ]
        for cp in copies:
            cp.wait()
        pltpu.sync_copy(rows_v, out_hbm.at[wid])

    return gather, nw, nch, ch


def kernel(x, codebook):
    idx = _argmin_call(x, codebook)
    gather, nw, nch, ch = _make_gather()
    quantized = gather(codebook, idx.reshape(nw, nch, ch))
    return quantized.reshape(N, C), idx
